# Initial kernel scaffold; baseline (speedup 1.0000x reference)
import jax
import jax.numpy as jnp
from jax.experimental import pallas as pl


def kernel(queries, keys, values):
    raise NotImplementedError("write your pallas kernel here")


# speedup vs baseline: 3.0974x; 3.0974x over previous
import functools
import math

import jax
import jax.numpy as jnp
import numpy as np
from jax.experimental import pallas as pl

_FACTOR = 5
_PAD = 64
_HIGHEST = jax.lax.Precision.HIGHEST


@functools.lru_cache(maxsize=None)
def _count_table(L_Q: int, L_K: int, sample_k: int):
    cpu = jax.local_devices(backend="cpu")[0]
    with jax.ensure_compile_time_eval(), jax.default_device(cpu):
        idx = np.asarray(
            jax.random.randint(jax.random.key(42), (L_Q, sample_k), 0, L_K)
        )
    cnt = np.zeros((L_Q, L_K), np.float32)
    np.add.at(cnt, (np.arange(L_Q)[:, None], idx), 1.0)
    return cnt


def _mstats_body(q_ref, k_ref, cnt_ref, m_ref):
    H = q_ref.shape[0]
    L_K = k_ref.shape[1]
    cnt = cnt_ref[...]
    mask = cnt > 0.0
    for h in range(H):
        s = jax.lax.dot_general(
            q_ref[h].astype(jnp.bfloat16), k_ref[h].astype(jnp.bfloat16),
            (((1,), (1,)), ((), ())),
            preferred_element_type=jnp.float32)
        smax = jnp.max(jnp.where(mask, s, -jnp.inf), axis=1)
        ssum = jnp.sum(s * cnt, axis=1)
        m_ref[h, :] = smax - ssum * (1.0 / L_K)


def _compute_m(q3, k3, cnt):
    H, L, D = q3.shape
    QB = 256
    return pl.pallas_call(
        _mstats_body,
        grid=(L // QB,),
        in_specs=[
            pl.BlockSpec((H, QB, D), lambda i: (0, i, 0)),
            pl.BlockSpec((H, L, D), lambda i: (0, 0, 0)),
            pl.BlockSpec((QB, L), lambda i: (i, 0)),
        ],
        out_specs=pl.BlockSpec((H, QB), lambda i: (0, i)),
        out_shape=jax.ShapeDtypeStruct((H, L), jnp.float32),
    )(q3, k3, cnt)


def _attend_body(n_top, m_ref, q_ref, k_ref, v_ref, ctx_ref, upd_ref, sel_ref):
    L, D = k_ref.shape[1], k_ref.shape[2]
    h = pl.program_id(0)
    m = m_ref[0]
    lanes = jax.lax.broadcasted_iota(jnp.int32, (1, L), 1)
    big = jnp.int32(L)

    vmin = jnp.min(m, axis=1, keepdims=True)
    minpos = jnp.min(jnp.where(m == vmin, lanes, big), axis=1, keepdims=True)
    lane_pad = jax.lax.broadcasted_iota(jnp.int32, (1, _PAD), 1)
    posvec0 = jnp.broadcast_to(minpos, (1, _PAD)).astype(jnp.int32)

    def step(u, carry):
        mc, pv = carry
        vmax = jnp.max(mc, axis=1, keepdims=True)
        sp = jnp.min(jnp.where(mc == vmax, lanes, big), axis=1, keepdims=True)
        pv = jnp.where(lane_pad == u, sp, pv)
        mc = jnp.where(lanes == sp, -jnp.inf, mc)
        return mc, pv

    _, posvec = jax.lax.fori_loop(0, n_top, step, (m, posvec0))

    ii = jax.lax.broadcasted_iota(jnp.int32, (_PAD, _PAD), 0)
    jj = jax.lax.broadcasted_iota(jnp.int32, (_PAD, _PAD), 1)
    eye = (ii == jj).astype(jnp.float32)
    pv_col = jax.lax.dot_general(
        eye, posvec.astype(jnp.float32), (((1,), (1,)), ((), ())),
        preferred_element_type=jnp.float32, precision=_HIGHEST)
    pv_col_i = pv_col.astype(jnp.int32)
    lanes_i = jax.lax.broadcasted_iota(jnp.int32, (_PAD, L), 1)
    row_ok = jax.lax.broadcasted_iota(jnp.int32, (_PAD, L), 0) < n_top
    oh = jnp.where((pv_col_i == lanes_i) & row_ok, 1.0, 0.0)

    q = q_ref[0]
    k = k_ref[0]
    v = v_ref[0]
    qr = jax.lax.dot_general(oh, q, (((1,), (0,)), ((), ())),
                             preferred_element_type=jnp.float32,
                             precision=_HIGHEST)
    qk = jax.lax.dot_general(qr, k, (((1,), (1,)), ((), ())),
                             preferred_element_type=jnp.float32,
                             precision=_HIGHEST)
    s = qk * (1.0 / math.sqrt(D))
    smax = jnp.max(s, axis=1, keepdims=True)
    e = jnp.exp(s - smax)
    attn = e / jnp.sum(e, axis=1, keepdims=True)
    upd = jax.lax.dot_general(attn, v, (((1,), (0,)), ((), ())),
                              preferred_element_type=jnp.float32,
                              precision=_HIGHEST)
    vmean = jnp.sum(v, axis=0, keepdims=True) * (1.0 / L)

    ones_col = jnp.ones((_PAD, 1), jnp.float32)
    ind_col = jax.lax.dot_general(oh, ones_col, (((0,), (0,)), ((), ())),
                                  preferred_element_type=jnp.float32,
                                  precision=_HIGHEST)
    scattered = jax.lax.dot_general(oh, upd, (((0,), (0,)), ((), ())),
                                    preferred_element_type=jnp.float32,
                                    precision=_HIGHEST)
    ctx_ref[...] = scattered + (1.0 - ind_col) * vmean

    rowi = jax.lax.broadcasted_iota(jnp.int32, (_PAD, 1), 0)
    upd_ref[0] = jnp.where(rowi < n_top, upd, vmean)
    sel_ref[0] = posvec * jnp.int32(pl.num_programs(0)) + h


def _attend(m, q3, k3, v3, n_top):
    H, L, D = q3.shape
    m3 = m.reshape(H, 1, L)
    return pl.pallas_call(
        functools.partial(_attend_body, n_top),
        grid=(H,),
        in_specs=[
            pl.BlockSpec((1, 1, L), lambda h: (h, 0, 0)),
            pl.BlockSpec((1, L, D), lambda h: (h, 0, 0)),
            pl.BlockSpec((1, L, D), lambda h: (h, 0, 0)),
            pl.BlockSpec((1, L, D), lambda h: (h, 0, 0)),
        ],
        out_specs=[
            pl.BlockSpec((L, D), lambda h: (0, h)),
            pl.BlockSpec((1, _PAD, D), lambda h: (h, 0, 0)),
            pl.BlockSpec((1, 1, _PAD), lambda h: (h, 0, 0)),
        ],
        out_shape=[
            jax.ShapeDtypeStruct((L, H * D), jnp.float32),
            jax.ShapeDtypeStruct((H, _PAD, D), jnp.float32),
            jax.ShapeDtypeStruct((H, 1, _PAD), jnp.int32),
        ],
    )(m3, q3, k3, v3)


def kernel(queries, keys, values):
    B, L_Q, H, D = queries.shape
    L_K = keys.shape[1]
    assert B == 1
    sample_k = max(1, min(_FACTOR * int(np.ceil(np.log(L_Q))), L_K))
    n_top = max(1, min(_FACTOR * int(np.ceil(np.log(L_K))), L_Q))
    assert n_top <= _PAD

    q3 = jnp.transpose(queries, (0, 2, 1, 3))[0]
    k3 = jnp.transpose(keys, (0, 2, 1, 3))[0]
    v3 = jnp.transpose(values, (0, 2, 1, 3))[0]

    cnt = _count_table(L_Q, L_K, sample_k)
    m = _compute_m(q3, k3, cnt)
    ctx2d, _upd, _sel = _attend(m, q3, k3, v3, n_top)
    return ctx2d.reshape(1, L_Q, H, D)


# speedup vs baseline: 3.2284x; 1.0423x over previous
import functools
import math

import jax
import jax.numpy as jnp
import numpy as np
from jax.experimental import pallas as pl
from jax.experimental.pallas import tpu as pltpu

_FACTOR = 5
_PAD = 64
_HIGHEST = jax.lax.Precision.HIGHEST


@functools.lru_cache(maxsize=None)
def _count_table(L_Q: int, L_K: int, sample_k: int):
    cpu = jax.local_devices(backend="cpu")[0]
    with jax.ensure_compile_time_eval(), jax.default_device(cpu):
        idx = np.asarray(
            jax.random.randint(jax.random.key(42), (L_Q, sample_k), 0, L_K)
        )
    cnt = np.zeros((L_Q, L_K), np.float32)
    np.add.at(cnt, (np.arange(L_Q)[:, None], idx), 1.0)
    return cnt


def _mstats_body(n_top, q_ref, k_ref, cnt_ref, pos_ref, m_acc):
    H = q_ref.shape[1]
    L_K = k_ref.shape[0]
    QB = q_ref.shape[0]
    i = pl.program_id(0)
    cnt = cnt_ref[...]
    mask = cnt > 0.0
    for h in range(H):
        s = jax.lax.dot_general(
            q_ref[:, h, :].astype(jnp.bfloat16),
            k_ref[:, h, :].astype(jnp.bfloat16),
            (((1,), (1,)), ((), ())),
            preferred_element_type=jnp.float32)
        smax = jnp.max(jnp.where(mask, s, -jnp.inf), axis=1)
        ssum = jnp.sum(s * cnt, axis=1)
        m_acc[h, pl.ds(i * QB, QB)] = smax - ssum * (1.0 / L_K)

    @pl.when(i == pl.num_programs(0) - 1)
    def _topk():
        m = m_acc[...]
        L = m.shape[1]
        lanes = jax.lax.broadcasted_iota(jnp.int32, (H, L), 1)
        big = jnp.int32(L)
        vmin = jnp.min(m, axis=1, keepdims=True)
        minpos = jnp.min(jnp.where(m == vmin, lanes, big), axis=1,
                         keepdims=True)
        lane_pad = jax.lax.broadcasted_iota(jnp.int32, (H, _PAD), 1)
        pv0 = jnp.broadcast_to(minpos, (H, _PAD)).astype(jnp.int32)

        def step(u, carry):
            mc, pv = carry
            vmax = jnp.max(mc, axis=1, keepdims=True)
            sp = jnp.min(jnp.where(mc == vmax, lanes, big), axis=1,
                         keepdims=True)
            pv = jnp.where(lane_pad == u, sp, pv)
            mc = jnp.where(lanes == sp, -jnp.inf, mc)
            return mc, pv

        _, pv = jax.lax.fori_loop(0, n_top, step, (m, pv0))
        pos_ref[...] = pv


def _compute_topk(q4, k4, cnt, n_top):
    L, H, D = q4.shape
    QB = 256
    return pl.pallas_call(
        functools.partial(_mstats_body, n_top),
        grid=(L // QB,),
        in_specs=[
            pl.BlockSpec((QB, H, D), lambda i: (i, 0, 0)),
            pl.BlockSpec((L, H, D), lambda i: (0, 0, 0)),
            pl.BlockSpec((QB, L), lambda i: (i, 0)),
        ],
        out_specs=pl.BlockSpec((H, _PAD), lambda i: (0, 0)),
        out_shape=jax.ShapeDtypeStruct((H, _PAD), jnp.int32),
        scratch_shapes=[pltpu.VMEM((H, L), jnp.float32)],
    )(q4, k4, cnt)


def _attend_body(n_top, pos_ref, q_ref, k_ref, v_ref, ctx_ref, upd_ref,
                 sel_ref):
    L, D = k_ref.shape[0], k_ref.shape[1]
    h = pl.program_id(0)
    posvec = pos_ref[0]

    ii = jax.lax.broadcasted_iota(jnp.int32, (_PAD, _PAD), 0)
    jj = jax.lax.broadcasted_iota(jnp.int32, (_PAD, _PAD), 1)
    eye = (ii == jj).astype(jnp.float32)
    pv_col = jax.lax.dot_general(
        eye, posvec.astype(jnp.float32), (((1,), (1,)), ((), ())),
        preferred_element_type=jnp.float32, precision=_HIGHEST)
    pv_col_i = pv_col.astype(jnp.int32)
    lanes_i = jax.lax.broadcasted_iota(jnp.int32, (_PAD, L), 1)
    row_ok = jax.lax.broadcasted_iota(jnp.int32, (_PAD, L), 0) < n_top
    oh = jnp.where((pv_col_i == lanes_i) & row_ok, 1.0, 0.0)

    q = q_ref[...]
    k = k_ref[...]
    v = v_ref[...]
    qr = jax.lax.dot_general(oh.astype(jnp.bfloat16), q.astype(jnp.bfloat16),
                             (((1,), (0,)), ((), ())),
                             preferred_element_type=jnp.float32)
    qk = jax.lax.dot_general(qr.astype(jnp.bfloat16), k.astype(jnp.bfloat16),
                             (((1,), (1,)), ((), ())),
                             preferred_element_type=jnp.float32)
    s = qk * (1.0 / math.sqrt(D))
    smax = jnp.max(s, axis=1, keepdims=True)
    e = jnp.exp(s - smax)
    attn = e / jnp.sum(e, axis=1, keepdims=True)
    upd = jax.lax.dot_general(attn.astype(jnp.bfloat16),
                              v.astype(jnp.bfloat16),
                              (((1,), (0,)), ((), ())),
                              preferred_element_type=jnp.float32)
    vmean = jnp.sum(v, axis=0, keepdims=True) * (1.0 / L)

    ones_col = jnp.ones((_PAD, 1), jnp.float32)
    ind_col = jax.lax.dot_general(oh, ones_col, (((0,), (0,)), ((), ())),
                                  preferred_element_type=jnp.float32,
                                  precision=_HIGHEST)
    scattered = jax.lax.dot_general(oh, upd, (((0,), (0,)), ((), ())),
                                    preferred_element_type=jnp.float32,
                                    precision=_HIGHEST)
    ctx_ref[...] = scattered + (1.0 - ind_col) * vmean

    rowi = jax.lax.broadcasted_iota(jnp.int32, (_PAD, 1), 0)
    upd_ref[0] = jnp.where(rowi < n_top, upd, vmean)
    sel_ref[0] = posvec * jnp.int32(pl.num_programs(0)) + h


def _attend(pos, q4, k4, v4, n_top):
    L, H, D = q4.shape
    pos3 = pos.reshape(H, 1, _PAD)
    q2 = q4.reshape(L, H * D)
    k2 = k4.reshape(L, H * D)
    v2 = v4.reshape(L, H * D)
    return pl.pallas_call(
        functools.partial(_attend_body, n_top),
        grid=(H,),
        in_specs=[
            pl.BlockSpec((1, 1, _PAD), lambda h: (h, 0, 0)),
            pl.BlockSpec((L, D), lambda h: (0, h)),
            pl.BlockSpec((L, D), lambda h: (0, h)),
            pl.BlockSpec((L, D), lambda h: (0, h)),
        ],
        out_specs=[
            pl.BlockSpec((L, D), lambda h: (0, h)),
            pl.BlockSpec((1, _PAD, D), lambda h: (h, 0, 0)),
            pl.BlockSpec((1, 1, _PAD), lambda h: (h, 0, 0)),
        ],
        out_shape=[
            jax.ShapeDtypeStruct((L, H * D), jnp.float32),
            jax.ShapeDtypeStruct((H, _PAD, D), jnp.float32),
            jax.ShapeDtypeStruct((H, 1, _PAD), jnp.int32),
        ],
    )(pos3, q2, k2, v2)


def kernel(queries, keys, values):
    B, L_Q, H, D = queries.shape
    L_K = keys.shape[1]
    assert B == 1
    sample_k = max(1, min(_FACTOR * int(np.ceil(np.log(L_Q))), L_K))
    n_top = max(1, min(_FACTOR * int(np.ceil(np.log(L_K))), L_Q))
    assert n_top <= _PAD

    q4 = queries[0]
    k4 = keys[0]
    v4 = values[0]

    cnt = _count_table(L_Q, L_K, sample_k)
    pos = _compute_topk(q4, k4, cnt, n_top)
    ctx2d, _upd, _sel = _attend(pos, q4, k4, v4, n_top)
    return ctx2d.reshape(1, L_Q, H, D)
